# baseline (speedup 1.0000x reference)
import jax
import jax.numpy as jnp
from jax.experimental import pallas as pl
from jax.experimental.pallas import tpu as pltpu

B = 4096
D = 1024
F = 65536
K = 32
BF = 512


def _mm_body(x_ref, w_ref, s_ref):
    w = w_ref[...]
    norm = jnp.sqrt(jnp.sum(w * w, axis=1, keepdims=True)) + 1e-6
    wn = w / norm
    s_ref[...] = jax.lax.dot_general(
        x_ref[...], wn, (((1,), (1,)), ((), ())),
        preferred_element_type=jnp.float32)


def kernel(x, dict_w):
    scores = pl.pallas_call(
        _mm_body,
        grid=(F // BF,),
        in_specs=[pl.BlockSpec((B, D), lambda j: (0, 0)),
                  pl.BlockSpec((BF, D), lambda j: (j, 0))],
        out_specs=pl.BlockSpec((B, BF), lambda j: (0, j)),
        out_shape=jax.ShapeDtypeStruct((B, F), jnp.float32),
    )(x, dict_w)
    vals, idx = jax.lax.top_k(scores, K)
    code = jnp.zeros((B, F), jnp.float32).at[
        jnp.arange(B)[:, None], idx].set(vals)
    return jax.nn.relu(code)


# speedup vs baseline: 11.8698x; 11.8698x over previous
import functools

import jax
import jax.numpy as jnp
from jax import lax
from jax.experimental import pallas as pl
from jax.experimental.pallas import tpu as pltpu
from jax.experimental.pallas import tpu_sc as plsc

B = 4096
D = 1024
F = 65536
K = 32
BF = 512
NCH = 4
C = BF // NCH
NCHUNK = F // C

NW = 32
RPW = B // NW
NEG = -3.0e38
CANDCAP = 192


def _mm_body(x_ref, w_ref, s_ref, cm_ref):
    w = w_ref[...]
    norm = jnp.sqrt(jnp.sum(w * w, axis=1, keepdims=True)) + 1e-6
    wn = w / norm
    s = jax.lax.dot_general(
        x_ref[...], wn, (((1,), (1,)), ((), ())),
        preferred_element_type=jnp.float32)
    s_ref[...] = s
    cm_ref[0] = jnp.max(s.reshape(B, NCH, C), axis=-1)


def _sort16(k):
    return lax.rev(jnp.sort(k), (0,))


def _merge16k(ak, bk):
    rbk = lax.rev(bk, (0,))
    hik = jnp.maximum(ak, rbk)
    lok = jnp.minimum(ak, rbk)
    return _sort16(hik), _sort16(lok)


def _merge16kv(ak, av, bk, bv):
    rbk = lax.rev(bk, (0,))
    rbv = lax.rev(bv, (0,))
    m = ak >= rbk
    hik = jnp.where(m, ak, rbk)
    hiv = jnp.where(m, av, rbv)
    lok = jnp.where(m, rbk, ak)
    lov = jnp.where(m, rbv, av)
    hik, hiv = plsc.sort_key_val(hik, hiv, descending=True)
    lok, lov = plsc.sort_key_val(lok, lov, descending=True)
    return hik, hiv, lok, lov


def _sc_body(srows, cmax, thr_hbm, cbuf, gbuf, cand, thr, dsem, gsem):
    nc = 2
    wid = lax.axis_index("s") * nc + lax.axis_index("c")
    base = wid * RPW
    lanes = lax.iota(jnp.int32, 16)
    lane0 = lanes == 0

    pltpu.async_copy(cmax.at[base], cbuf.at[0], dsem)

    def row_body(r, _):
        row = base + r
        buf = r & 1
        @pl.when(r + 1 < RPW)
        def _():
            pltpu.async_copy(cmax.at[row + 1], cbuf.at[(r + 1) & 1], dsem)
        pltpu.make_async_copy(cmax.at[row], cbuf.at[buf], dsem).wait()

        ck0 = cbuf[buf, pl.ds(0, 16)]
        t0k, t0v = plsc.sort_key_val(ck0, lanes, descending=True)
        t1k = jnp.full((16,), NEG, jnp.float32)
        t1v = jnp.zeros((16,), jnp.int32)

        def amerge(i, carry):
            t0k, t0v, t1k, t1v = carry
            nk = cbuf[buf, pl.ds(i * 16, 16)]
            nv = lanes + i * 16
            nk, nv = plsc.sort_key_val(nk, nv, descending=True)
            hk, hv, _, _ = _merge16kv(t1k, t1v, nk, nv)
            t0k, t0v, t1k, t1v = _merge16kv(t0k, t0v, hk, hv)
            return t0k, t0v, t1k, t1v

        t0k, t0v, t1k, t1v = lax.fori_loop(
            1, NCHUNK // 16, amerge, (t0k, t0v, t1k, t1v))
        tau = jnp.min(t1k)

        rbase = row * NCHUNK
        d0 = pltpu.async_copy(srows.at[t0v + rbase], gbuf.at[pl.ds(0, 16)],
                              gsem)
        d1 = pltpu.async_copy(srows.at[t1v + rbase], gbuf.at[pl.ds(16, 16)],
                              gsem)

        for j in range(CANDCAP // 16):
            cand[pl.ds(j * 16, 16)] = jnp.full((16,), NEG, jnp.float32)
        d0.wait()
        d1.wait()

        def cscan(i, cnt):
            v = gbuf[i >> 3, pl.ds((i & 7) * 16, 16)]
            m = v >= tau
            mi = m.astype(jnp.int32)
            excl = plsc.cumsum(mi) - mi
            idxs = jnp.minimum(cnt + excl, CANDCAP - 1)
            plsc.store_scatter(cand, [idxs], v, mask=m)
            return cnt + plsc.all_reduce_population_count(m)

        lax.fori_loop(0, K * (C // 16), cscan,
                      jnp.zeros((16,), jnp.int32))

        u0 = _sort16(cand[pl.ds(0, 16)])
        u1 = jnp.full((16,), NEG, jnp.float32)

        def dmerge(j, carry):
            u0, u1 = carry
            n = _sort16(cand[pl.ds(j * 16, 16)])
            h, _ = _merge16k(u1, n)
            return _merge16k(u0, h)

        u0, u1 = lax.fori_loop(1, CANDCAP // 16, dmerge, (u0, u1))
        t = jnp.min(u1)
        plsc.store_scatter(thr, [jnp.full((16,), r, jnp.int32)],
                           jnp.full((16,), t, jnp.float32), mask=lane0)
        return 0

    lax.fori_loop(0, RPW, row_body, 0)
    pltpu.sync_copy(thr, thr_hbm.at[pl.ds(base, RPW)])


def _mask_body(s_ref, t_ref, o_ref):
    s = s_ref[...]
    o_ref[...] = jnp.where(s >= t_ref[...], jnp.maximum(s, 0.0), 0.0)


def kernel(x, dict_w):
    scores, cmax3 = pl.pallas_call(
        _mm_body,
        grid=(F // BF,),
        in_specs=[pl.BlockSpec((B, D), lambda j: (0, 0)),
                  pl.BlockSpec((BF, D), lambda j: (j, 0))],
        out_specs=[pl.BlockSpec((B, BF), lambda j: (0, j)),
                   pl.BlockSpec((1, B, NCH), lambda j: (j, 0, 0))],
        out_shape=[jax.ShapeDtypeStruct((B, F), jnp.float32),
                   jax.ShapeDtypeStruct((F // BF, B, NCH), jnp.float32)],
    )(x, dict_w)

    cmax = cmax3.transpose(1, 0, 2).reshape(B, NCHUNK)
    srows = scores.reshape(B * NCHUNK, C)

    mesh = plsc.VectorSubcoreMesh(core_axis_name="c", subcore_axis_name="s")
    thresh = pl.kernel(
        _sc_body,
        out_type=jax.ShapeDtypeStruct((B,), jnp.float32),
        mesh=mesh,
        compiler_params=pltpu.CompilerParams(needs_layout_passes=False),
        scratch_types=[
            pltpu.VMEM((2, NCHUNK), jnp.float32),
            pltpu.VMEM((K, C), jnp.float32),
            pltpu.VMEM((CANDCAP,), jnp.float32),
            pltpu.VMEM((RPW,), jnp.float32),
            pltpu.SemaphoreType.DMA,
            pltpu.SemaphoreType.DMA,
        ],
    )(srows, cmax)

    return pl.pallas_call(
        _mask_body,
        grid=(F // BF,),
        in_specs=[pl.BlockSpec((B, BF), lambda j: (0, j)),
                  pl.BlockSpec((B, 1), lambda j: (0, 0))],
        out_specs=pl.BlockSpec((B, BF), lambda j: (0, j)),
        out_shape=jax.ShapeDtypeStruct((B, F), jnp.float32),
    )(scores, thresh.reshape(B, 1))
